# baseline (speedup 1.0000x reference)
import functools

import jax
import jax.numpy as jnp
from jax import lax
from jax.experimental import pallas as pl
from jax.experimental.pallas import tpu as pltpu
from jax.experimental.pallas import tpu_sc as plsc

_NC, _NS, _L = 2, 16, 16
_NW = _NC * _NS

_WRD = 16
_CHN = 8
_ROWS_PER_CHUNK = _CHN * _WRD


def _emb_body(idx_hbm, embed_hbm, out_hbm, idx_v, rows_v, out_v, sem):
    npw = out_v.shape[0]
    nchunks = npw // _CHN
    d = embed_hbm.shape[1]
    nblk = d // _L
    wid = lax.axis_index("s") * _NC + lax.axis_index("c")
    pltpu.sync_copy(idx_hbm.at[pl.ds(wid * nchunks, nchunks)], idx_v)

    def chunk(ci, _):
        pltpu.async_copy(embed_hbm.at[idx_v.at[ci]], rows_v, sem).wait()

        def node(j, _):
            for cb in range(nblk):
                acc = rows_v[j * _WRD, pl.ds(cb * _L, _L)]
                for r in range(1, _WRD):
                    acc = acc + rows_v[j * _WRD + r, pl.ds(cb * _L, _L)]
                out_v[ci * _CHN + j, pl.ds(cb * _L, _L)] = acc * (1.0 / _WRD)
            return 0

        lax.fori_loop(0, _CHN, node, 0)
        return 0

    lax.fori_loop(0, nchunks, chunk, 0)
    pltpu.sync_copy(out_v, out_hbm.at[pl.ds(wid * npw, npw)])


def _embedding_mean(x_index, embed):
    n, wrd = x_index.shape
    d = embed.shape[1]
    npw = -(-n // (_NW * _CHN)) * _CHN
    b = npw * _NW
    idx = jnp.pad(x_index, ((0, b - n), (0, 0))).reshape(-1, _ROWS_PER_CHUNK)
    mesh = plsc.VectorSubcoreMesh(core_axis_name="c", subcore_axis_name="s")
    emb_k = functools.partial(
        pl.kernel,
        mesh=mesh,
        out_type=jax.ShapeDtypeStruct((b, d), jnp.float32),
        scratch_types=[
            pltpu.VMEM((npw // _CHN, _ROWS_PER_CHUNK), jnp.int32),
            pltpu.VMEM((_ROWS_PER_CHUNK, d), jnp.float32),
            pltpu.VMEM((npw, d), jnp.float32),
            pltpu.SemaphoreType.DMA,
        ],
    )(_emb_body)
    return emb_k(idx, embed)


_B = 16


def _scan_body(seq_ref, flag_ref, node_emb_ref, w_ih_t_ref, b_ih_ref,
               w_hh_t_ref, b_hh_ref, weight_ref, wp_ref, out_w_t_ref,
               out_b_ref, out_ref, h_ref, gi_ref, gi_buf, topo_buf, temp_buf):
    n_nodes, hid = h_ref.shape
    n_steps = seq_ref.shape[1]
    n_emb = node_emb_ref.shape[0]

    h_ref[...] = jnp.zeros_like(h_ref)

    _PR = 128
    def gi_chunk(c, _):
        rows = node_emb_ref[pl.ds(c * _PR, _PR), :]
        gi_ref[pl.ds(c * _PR, _PR), :] = jnp.dot(
            rows, w_ih_t_ref[...],
            preferred_element_type=jnp.float32) + b_ih_ref[...]
        return 0

    jax.lax.fori_loop(0, n_emb // _PR, gi_chunk, 0)

    def gru_att(gi, temp, topo):
        gh = jnp.dot(temp, w_hh_t_ref[...],
                     preferred_element_type=jnp.float32) + b_hh_ref[...]
        i_r, i_z, i_n = (gi[:, :hid], gi[:, hid:2 * hid], gi[:, 2 * hid:])
        h_r, h_z, h_n = (gh[:, :hid], gh[:, hid:2 * hid], gh[:, 2 * hid:])
        r = jax.nn.sigmoid(i_r + h_r)
        z = jax.nn.sigmoid(i_z + h_z)
        n = jnp.tanh(i_n + r * h_n)
        h1 = (1.0 - z) * n + z * temp
        m = topo.shape[0]
        u = jnp.tanh(jnp.dot(jnp.concatenate([topo, h1], axis=0),
                             weight_ref[...],
                             preferred_element_type=jnp.float32))
        a = jnp.sum(u * wp_ref[...], axis=1, keepdims=True)
        s = jax.nn.sigmoid(a[m:] - a[:m])
        return topo + s * (h1 - topo)

    def step(i, _):
        nid = seq_ref[0, i]
        parent = seq_ref[1, i]
        prior = seq_ref[2, i]
        h_new = gru_att(gi_ref[pl.ds(nid, 1), :],
                        h_ref[pl.ds(prior, 1), :],
                        h_ref[pl.ds(parent, 1), :])
        h_ref[pl.ds(nid, 1), :] = h_new
        return 0

    nb = n_steps // _B

    def gather_gi(base):
        return jnp.concatenate(
            [gi_ref[pl.ds(seq_ref[0, base + j], 1), :] for j in range(_B)],
            axis=0)

    def gather_h(row, base):
        return jnp.concatenate(
            [h_ref[pl.ds(seq_ref[row, base + j], 1), :] for j in range(_B)],
            axis=0)

    gi_buf[...] = gather_gi(0)
    topo_buf[...] = gather_h(1, 0)
    temp_buf[...] = gather_h(2, 0)

    def batch(bi, _):
        base = bi * _B
        bp = jnp.minimum(bi + 1, nb - 1)
        pbase = bp * _B
        gi = gi_buf[...]
        topo = topo_buf[...]
        temp = temp_buf[...]
        pre_gi = gather_gi(pbase)
        pre_topo = gather_h(1, pbase)
        pre_temp = gather_h(2, pbase)

        @pl.when(flag_ref[0, bi] == 0)
        def _fast():
            h_new = gru_att(gi, temp, topo)
            for j in range(_B):
                h_ref[pl.ds(seq_ref[0, base + j], 1), :] = h_new[j:j + 1, :]

        @pl.when(flag_ref[0, bi] != 0)
        def _slow():
            jax.lax.fori_loop(base, base + _B, step, 0)

        gi_buf[...] = pre_gi
        topo_buf[...] = pre_topo
        temp_buf[...] = pre_temp

        @pl.when(flag_ref[1, bi] != 0)
        def _regather():
            topo_buf[...] = gather_h(1, pbase)
            temp_buf[...] = gather_h(2, pbase)

        return 0

    jax.lax.fori_loop(0, nb, batch, 0)

    h_last = h_ref[pl.ds(n_nodes - 1, 1), :]
    logits = jnp.dot(h_last, out_w_t_ref[...],
                     preferred_element_type=jnp.float32) + out_b_ref[...]
    lm = logits - jnp.max(logits)
    out_ref[...] = lm - jnp.log(jnp.sum(jnp.exp(lm)))


def _batch_conflict_flags(seqs):
    nb = seqs.shape[1] // _B
    nid = seqs[0].reshape(nb, _B)
    par = seqs[1].reshape(nb, _B)
    pri = seqs[2].reshape(nb, _B)
    wr = nid[:, :, None]
    rd = (wr == par[:, None, :]) | (wr == pri[:, None, :])
    order = jnp.tril(jnp.ones((_B, _B), jnp.bool_), -1).T
    intra = jnp.any(rd & order[None], axis=(1, 2))
    wr_p = nid[:-1][:, :, None]
    cross = jnp.any((wr_p == par[1:][:, None, :]) |
                    (wr_p == pri[1:][:, None, :]), axis=(1, 2))
    stale = jnp.concatenate([cross, jnp.zeros((1,), jnp.bool_)])
    return jnp.stack([intra, stale]).astype(jnp.int32)


def _rnn_scan(seqs, flags, node_emb, W_ih, W_hh, b_ih, b_hh, weight,
              weight_proj, out_W, out_b, *, interpret=False):
    n_nodes, hid = seqs.shape[1], weight.shape[0]
    nclass = out_W.shape[0]
    grid_spec = pltpu.PrefetchScalarGridSpec(
        num_scalar_prefetch=2,
        grid=(1,),
        in_specs=[
            pl.BlockSpec(node_emb.shape, lambda i, s, f: (0, 0)),
            pl.BlockSpec((hid, 3 * hid), lambda i, s, f: (0, 0)),
            pl.BlockSpec((1, 3 * hid), lambda i, s, f: (0, 0)),
            pl.BlockSpec((hid, 3 * hid), lambda i, s, f: (0, 0)),
            pl.BlockSpec((1, 3 * hid), lambda i, s, f: (0, 0)),
            pl.BlockSpec((hid, hid), lambda i, s, f: (0, 0)),
            pl.BlockSpec((1, hid), lambda i, s, f: (0, 0)),
            pl.BlockSpec((hid, nclass), lambda i, s, f: (0, 0)),
            pl.BlockSpec((1, nclass), lambda i, s, f: (0, 0)),
        ],
        out_specs=pl.BlockSpec((1, nclass), lambda i, s, f: (0, 0)),
        scratch_shapes=[
            pltpu.VMEM((n_nodes, hid), jnp.float32),
            pltpu.VMEM((node_emb.shape[0], 3 * hid), jnp.float32),
            pltpu.VMEM((_B, 3 * hid), jnp.float32),
            pltpu.VMEM((_B, hid), jnp.float32),
            pltpu.VMEM((_B, hid), jnp.float32),
        ],
    )
    return pl.pallas_call(
        _scan_body,
        grid_spec=grid_spec,
        out_shape=jax.ShapeDtypeStruct((1, nclass), jnp.float32),
        interpret=interpret,
    )(seqs, flags, node_emb, W_ih.T, b_ih.reshape(1, -1), W_hh.T,
      b_hh.reshape(1, -1), weight, weight_proj.reshape(1, -1),
      out_W.T, out_b.reshape(1, -1))


def kernel(x_index, sequences, embed, weight, weight_proj, W_ih, W_hh, b_ih,
           b_hh, out_W, out_b):
    node_emb = _embedding_mean(x_index, embed)
    seqs = sequences[:, :, 0].T
    flags = _batch_conflict_flags(seqs)
    return _rnn_scan(seqs, flags, node_emb, W_ih, W_hh, b_ih, b_hh, weight,
                     weight_proj, out_W, out_b)


# speedup vs baseline: 1.0772x; 1.0772x over previous
import functools

import jax
import jax.numpy as jnp
from jax import lax
from jax.experimental import pallas as pl
from jax.experimental.pallas import tpu as pltpu
from jax.experimental.pallas import tpu_sc as plsc

_NC, _NS, _L = 2, 16, 16
_NW = _NC * _NS

_WRD = 16
_CHN = 8
_ROWS_PER_CHUNK = _CHN * _WRD


def _emb_body(idx_hbm, embed_hbm, out_hbm, idx_v, rows_v, out_v, sem):
    npw = out_v.shape[0]
    nchunks = npw // _CHN
    d = embed_hbm.shape[1]
    nblk = d // _L
    wid = lax.axis_index("s") * _NC + lax.axis_index("c")
    pltpu.sync_copy(idx_hbm.at[pl.ds(wid * nchunks, nchunks)], idx_v)

    def chunk(ci, _):
        pltpu.async_copy(embed_hbm.at[idx_v.at[ci]], rows_v, sem).wait()

        def node(j, _):
            for cb in range(nblk):
                acc = rows_v[j * _WRD, pl.ds(cb * _L, _L)]
                for r in range(1, _WRD):
                    acc = acc + rows_v[j * _WRD + r, pl.ds(cb * _L, _L)]
                out_v[ci * _CHN + j, pl.ds(cb * _L, _L)] = acc * (1.0 / _WRD)
            return 0

        lax.fori_loop(0, _CHN, node, 0)
        return 0

    lax.fori_loop(0, nchunks, chunk, 0)
    pltpu.sync_copy(out_v, out_hbm.at[pl.ds(wid * npw, npw)])


def _embedding_mean(x_index, embed):
    n, wrd = x_index.shape
    d = embed.shape[1]
    npw = -(-n // (_NW * _CHN)) * _CHN
    b = npw * _NW
    idx = jnp.pad(x_index, ((0, b - n), (0, 0))).reshape(-1, _ROWS_PER_CHUNK)
    mesh = plsc.VectorSubcoreMesh(core_axis_name="c", subcore_axis_name="s")
    emb_k = functools.partial(
        pl.kernel,
        mesh=mesh,
        out_type=jax.ShapeDtypeStruct((b, d), jnp.float32),
        scratch_types=[
            pltpu.VMEM((npw // _CHN, _ROWS_PER_CHUNK), jnp.int32),
            pltpu.VMEM((_ROWS_PER_CHUNK, d), jnp.float32),
            pltpu.VMEM((npw, d), jnp.float32),
            pltpu.SemaphoreType.DMA,
        ],
    )(_emb_body)
    return emb_k(idx, embed)


_B = 16


def _scan_body(seq_ref, flag_ref, node_emb_ref, w_ih_t_ref, b_ih_ref,
               w_hh_t_ref, b_hh_ref, weight_ref, wp_ref, out_w_t_ref,
               out_b_ref, out_ref, h_ref, gi_ref, gi_buf, topo_buf, temp_buf,
               hn_buf):
    n_nodes, hid = h_ref.shape
    n_steps = seq_ref.shape[1]
    n_emb = node_emb_ref.shape[0]

    h_ref[...] = jnp.zeros_like(h_ref)

    _PR = 128
    def gi_chunk(c, _):
        rows = node_emb_ref[pl.ds(c * _PR, _PR), :]
        gi_ref[pl.ds(c * _PR, _PR), :] = jnp.dot(
            rows, w_ih_t_ref[...],
            preferred_element_type=jnp.float32) + b_ih_ref[...]
        return 0

    jax.lax.fori_loop(0, n_emb // _PR, gi_chunk, 0)

    def gru_att(gi, temp, topo):
        gh = jnp.dot(temp, w_hh_t_ref[...],
                     preferred_element_type=jnp.float32) + b_hh_ref[...]
        i_r, i_z, i_n = (gi[:, :hid], gi[:, hid:2 * hid], gi[:, 2 * hid:])
        h_r, h_z, h_n = (gh[:, :hid], gh[:, hid:2 * hid], gh[:, 2 * hid:])
        r = jax.nn.sigmoid(i_r + h_r)
        z = jax.nn.sigmoid(i_z + h_z)
        n = jnp.tanh(i_n + r * h_n)
        h1 = (1.0 - z) * n + z * temp
        m = topo.shape[0]
        u = jnp.tanh(jnp.dot(jnp.concatenate([topo, h1], axis=0),
                             weight_ref[...],
                             preferred_element_type=jnp.float32))
        a = jnp.sum(u * wp_ref[...], axis=1, keepdims=True)
        s = jax.nn.sigmoid(a[m:] - a[:m])
        return topo + s * (h1 - topo)

    def step(i, _):
        nid = seq_ref[0, i]
        parent = seq_ref[1, i]
        prior = seq_ref[2, i]
        h_new = gru_att(gi_ref[pl.ds(nid, 1), :],
                        h_ref[pl.ds(prior, 1), :],
                        h_ref[pl.ds(parent, 1), :])
        h_ref[pl.ds(nid, 1), :] = h_new
        return 0

    nb = n_steps // _B

    def batch(bi, _):
        base = bi * _B

        @pl.when(flag_ref[0, bi] == 0)
        def _fast():
            for j in range(_B):
                gi_buf[pl.ds(j, 1), :] = gi_ref[
                    pl.ds(seq_ref[0, base + j], 1), :]
                topo_buf[pl.ds(j, 1), :] = h_ref[
                    pl.ds(seq_ref[1, base + j], 1), :]
                temp_buf[pl.ds(j, 1), :] = h_ref[
                    pl.ds(seq_ref[2, base + j], 1), :]
            h_new = gru_att(gi_buf[...], temp_buf[...], topo_buf[...])
            hn_buf[...] = h_new
            for j in range(_B):
                h_ref[pl.ds(seq_ref[0, base + j], 1), :] = hn_buf[
                    pl.ds(j, 1), :]

        @pl.when(flag_ref[0, bi] != 0)
        def _slow():
            jax.lax.fori_loop(base, base + _B, step, 0)

        return 0

    jax.lax.fori_loop(0, nb, batch, 0)

    h_last = h_ref[pl.ds(n_nodes - 1, 1), :]
    logits = jnp.dot(h_last, out_w_t_ref[...],
                     preferred_element_type=jnp.float32) + out_b_ref[...]
    lm = logits - jnp.max(logits)
    out_ref[...] = lm - jnp.log(jnp.sum(jnp.exp(lm)))


def _batch_conflict_flags(seqs):
    nb = seqs.shape[1] // _B
    nid = seqs[0].reshape(nb, _B)
    par = seqs[1].reshape(nb, _B)
    pri = seqs[2].reshape(nb, _B)
    wr = nid[:, :, None]
    rd = (wr == par[:, None, :]) | (wr == pri[:, None, :])
    order = jnp.tril(jnp.ones((_B, _B), jnp.bool_), -1).T
    intra = jnp.any(rd & order[None], axis=(1, 2))
    wr_p = nid[:-1][:, :, None]
    cross = jnp.any((wr_p == par[1:][:, None, :]) |
                    (wr_p == pri[1:][:, None, :]), axis=(1, 2))
    stale = jnp.concatenate([cross, jnp.zeros((1,), jnp.bool_)])
    return jnp.stack([intra, stale]).astype(jnp.int32)


def _rnn_scan(seqs, flags, node_emb, W_ih, W_hh, b_ih, b_hh, weight,
              weight_proj, out_W, out_b, *, interpret=False):
    n_nodes, hid = seqs.shape[1], weight.shape[0]
    nclass = out_W.shape[0]
    grid_spec = pltpu.PrefetchScalarGridSpec(
        num_scalar_prefetch=2,
        grid=(1,),
        in_specs=[
            pl.BlockSpec(node_emb.shape, lambda i, s, f: (0, 0)),
            pl.BlockSpec((hid, 3 * hid), lambda i, s, f: (0, 0)),
            pl.BlockSpec((1, 3 * hid), lambda i, s, f: (0, 0)),
            pl.BlockSpec((hid, 3 * hid), lambda i, s, f: (0, 0)),
            pl.BlockSpec((1, 3 * hid), lambda i, s, f: (0, 0)),
            pl.BlockSpec((hid, hid), lambda i, s, f: (0, 0)),
            pl.BlockSpec((1, hid), lambda i, s, f: (0, 0)),
            pl.BlockSpec((hid, nclass), lambda i, s, f: (0, 0)),
            pl.BlockSpec((1, nclass), lambda i, s, f: (0, 0)),
        ],
        out_specs=pl.BlockSpec((1, nclass), lambda i, s, f: (0, 0)),
        scratch_shapes=[
            pltpu.VMEM((n_nodes, hid), jnp.float32),
            pltpu.VMEM((node_emb.shape[0], 3 * hid), jnp.float32),
            pltpu.VMEM((_B, 3 * hid), jnp.float32),
            pltpu.VMEM((_B, hid), jnp.float32),
            pltpu.VMEM((_B, hid), jnp.float32),
            pltpu.VMEM((_B, hid), jnp.float32),
        ],
    )
    return pl.pallas_call(
        _scan_body,
        grid_spec=grid_spec,
        out_shape=jax.ShapeDtypeStruct((1, nclass), jnp.float32),
        interpret=interpret,
    )(seqs, flags, node_emb, W_ih.T, b_ih.reshape(1, -1), W_hh.T,
      b_hh.reshape(1, -1), weight, weight_proj.reshape(1, -1),
      out_W.T, out_b.reshape(1, -1))


def kernel(x_index, sequences, embed, weight, weight_proj, W_ih, W_hh, b_ih,
           b_hh, out_W, out_b):
    node_emb = _embedding_mean(x_index, embed)
    seqs = sequences[:, :, 0].T
    flags = _batch_conflict_flags(seqs)
    return _rnn_scan(seqs, flags, node_emb, W_ih, W_hh, b_ih, b_hh, weight,
                     weight_proj, out_W, out_b)


# speedup vs baseline: 1.9540x; 1.8140x over previous
import functools

import jax
import jax.numpy as jnp
from jax import lax
from jax.experimental import pallas as pl
from jax.experimental.pallas import tpu as pltpu
from jax.experimental.pallas import tpu_sc as plsc

_NC, _NS, _L = 2, 16, 16
_NW = _NC * _NS

_WRD = 16
_CHN = 16
_GROWS = 128


def _emb_accum(rows_v, out_v, out_hbm, wid, ci, npw):
    nblk = rows_v.shape[1] // _L

    def node(j, _):
        base = j * _WRD
        for cb in range(nblk):
            sl = pl.ds(cb * _L, _L)
            r = [rows_v[base + t, sl] for t in range(_WRD)]
            while len(r) > 1:
                r = [r[2 * i] + r[2 * i + 1] for i in range(len(r) // 2)]
            out_v[j, sl] = r[0] * (1.0 / _WRD)
        return 0

    lax.fori_loop(0, _CHN, node, 0)
    pltpu.sync_copy(out_v, out_hbm.at[pl.ds(wid * npw + ci * _CHN, _CHN)])


def _emb_body(idx_hbm, embed_hbm, out_hbm, idx_v, rows0, rows1, out_v,
              sem0, sem1):
    npw = (idx_v.shape[0] // 2) * _CHN
    nchunks = npw // _CHN
    wid = lax.axis_index("s") * _NC + lax.axis_index("c")
    pltpu.sync_copy(idx_hbm.at[pl.ds(wid * 2 * nchunks, 2 * nchunks)], idx_v)

    def start(ci, rows_v, sem):
        pltpu.make_async_copy(embed_hbm.at[idx_v.at[2 * ci]],
                              rows_v.at[pl.ds(0, _GROWS)], sem).start()
        pltpu.make_async_copy(embed_hbm.at[idx_v.at[2 * ci + 1]],
                              rows_v.at[pl.ds(_GROWS, _GROWS)], sem).start()

    def drain(rows_v, sem):
        pltpu.make_async_copy(embed_hbm.at[idx_v.at[0]],
                              rows_v.at[pl.ds(0, _GROWS)], sem).wait()
        pltpu.make_async_copy(embed_hbm.at[idx_v.at[0]],
                              rows_v.at[pl.ds(_GROWS, _GROWS)], sem).wait()

    start(0, rows0, sem0)

    def outer(k, _):
        ci0 = 2 * k
        start(ci0 + 1, rows1, sem1)
        drain(rows0, sem0)
        _emb_accum(rows0, out_v, out_hbm, wid, ci0, npw)

        @pl.when(k < nchunks // 2 - 1)
        def _():
            start(ci0 + 2, rows0, sem0)

        drain(rows1, sem1)
        _emb_accum(rows1, out_v, out_hbm, wid, ci0 + 1, npw)
        return 0

    lax.fori_loop(0, nchunks // 2, outer, 0)


def _embedding_mean(x_index, embed):
    n, wrd = x_index.shape
    d = embed.shape[1]
    npw = -(-n // (_NW * 2 * _CHN)) * (2 * _CHN)
    b = npw * _NW
    idx = jnp.pad(x_index, ((0, b - n), (0, 0))).reshape(-1, _GROWS)
    mesh = plsc.VectorSubcoreMesh(core_axis_name="c", subcore_axis_name="s")
    emb_k = functools.partial(
        pl.kernel,
        mesh=mesh,
        out_type=jax.ShapeDtypeStruct((b, d), jnp.float32),
        scratch_types=[
            pltpu.VMEM((npw * _WRD // _GROWS, _GROWS), jnp.int32),
            pltpu.VMEM((2 * _GROWS, d), jnp.float32),
            pltpu.VMEM((2 * _GROWS, d), jnp.float32),
            pltpu.VMEM((_CHN, d), jnp.float32),
            pltpu.SemaphoreType.DMA,
            pltpu.SemaphoreType.DMA,
        ],
    )(_emb_body)
    return emb_k(idx, embed)


_B = 16


def _scan_body(seq_ref, flag_ref, node_emb_ref, w_ih_t_ref, b_ih_ref,
               w_hh_t_ref, b_hh_ref, weight_ref, wp_ref, out_w_t_ref,
               out_b_ref, out_ref, h_ref, gi_ref, gi_buf, topo_buf, temp_buf,
               hn_buf):
    n_nodes, hid = h_ref.shape
    n_steps = seq_ref.shape[1]
    n_emb = node_emb_ref.shape[0]

    h_ref[...] = jnp.zeros_like(h_ref)

    _PR = 128
    def gi_chunk(c, _):
        rows = node_emb_ref[pl.ds(c * _PR, _PR), :]
        gi_ref[pl.ds(c * _PR, _PR), :] = jnp.dot(
            rows, w_ih_t_ref[...],
            preferred_element_type=jnp.float32) + b_ih_ref[...]
        return 0

    jax.lax.fori_loop(0, n_emb // _PR, gi_chunk, 0)

    def gru_att(gi, temp, topo):
        gh = jnp.dot(temp, w_hh_t_ref[...],
                     preferred_element_type=jnp.float32) + b_hh_ref[...]
        i_r, i_z, i_n = (gi[:, :hid], gi[:, hid:2 * hid], gi[:, 2 * hid:])
        h_r, h_z, h_n = (gh[:, :hid], gh[:, hid:2 * hid], gh[:, 2 * hid:])
        r = jax.nn.sigmoid(i_r + h_r)
        z = jax.nn.sigmoid(i_z + h_z)
        n = jnp.tanh(i_n + r * h_n)
        h1 = (1.0 - z) * n + z * temp
        m = topo.shape[0]
        u = jnp.tanh(jnp.dot(jnp.concatenate([topo, h1], axis=0),
                             weight_ref[...],
                             preferred_element_type=jnp.float32))
        a = jnp.sum(u * wp_ref[...], axis=1, keepdims=True)
        s = jax.nn.sigmoid(a[m:] - a[:m])
        return topo + s * (h1 - topo)

    def step(i, _):
        nid = seq_ref[0, i]
        parent = seq_ref[1, i]
        prior = seq_ref[2, i]
        h_new = gru_att(gi_ref[pl.ds(nid, 1), :],
                        h_ref[pl.ds(prior, 1), :],
                        h_ref[pl.ds(parent, 1), :])
        h_ref[pl.ds(nid, 1), :] = h_new
        return 0

    nb = n_steps // _B

    def batch(bi, _):
        base = bi * _B

        @pl.when(flag_ref[0, bi] == 0)
        def _fast():
            for j in range(_B):
                gi_buf[pl.ds(j, 1), :] = gi_ref[
                    pl.ds(seq_ref[0, base + j], 1), :]
                topo_buf[pl.ds(j, 1), :] = h_ref[
                    pl.ds(seq_ref[1, base + j], 1), :]
                temp_buf[pl.ds(j, 1), :] = h_ref[
                    pl.ds(seq_ref[2, base + j], 1), :]
            h_new = gru_att(gi_buf[...], temp_buf[...], topo_buf[...])
            hn_buf[...] = h_new
            for j in range(_B):
                h_ref[pl.ds(seq_ref[0, base + j], 1), :] = hn_buf[
                    pl.ds(j, 1), :]

        @pl.when(flag_ref[0, bi] != 0)
        def _slow():
            jax.lax.fori_loop(base, base + _B, step, 0)

        return 0

    jax.lax.fori_loop(0, nb, batch, 0)

    h_last = h_ref[pl.ds(n_nodes - 1, 1), :]
    logits = jnp.dot(h_last, out_w_t_ref[...],
                     preferred_element_type=jnp.float32) + out_b_ref[...]
    lm = logits - jnp.max(logits)
    out_ref[...] = lm - jnp.log(jnp.sum(jnp.exp(lm)))


def _batch_conflict_flags(seqs):
    nb = seqs.shape[1] // _B
    nid = seqs[0].reshape(nb, _B)
    par = seqs[1].reshape(nb, _B)
    pri = seqs[2].reshape(nb, _B)
    wr = nid[:, :, None]
    rd = (wr == par[:, None, :]) | (wr == pri[:, None, :])
    order = jnp.tril(jnp.ones((_B, _B), jnp.bool_), -1).T
    intra = jnp.any(rd & order[None], axis=(1, 2))
    wr_p = nid[:-1][:, :, None]
    cross = jnp.any((wr_p == par[1:][:, None, :]) |
                    (wr_p == pri[1:][:, None, :]), axis=(1, 2))
    stale = jnp.concatenate([cross, jnp.zeros((1,), jnp.bool_)])
    return jnp.stack([intra, stale]).astype(jnp.int32)


def _rnn_scan(seqs, flags, node_emb, W_ih, W_hh, b_ih, b_hh, weight,
              weight_proj, out_W, out_b, *, interpret=False):
    n_nodes, hid = seqs.shape[1], weight.shape[0]
    nclass = out_W.shape[0]
    grid_spec = pltpu.PrefetchScalarGridSpec(
        num_scalar_prefetch=2,
        grid=(1,),
        in_specs=[
            pl.BlockSpec(node_emb.shape, lambda i, s, f: (0, 0)),
            pl.BlockSpec((hid, 3 * hid), lambda i, s, f: (0, 0)),
            pl.BlockSpec((1, 3 * hid), lambda i, s, f: (0, 0)),
            pl.BlockSpec((hid, 3 * hid), lambda i, s, f: (0, 0)),
            pl.BlockSpec((1, 3 * hid), lambda i, s, f: (0, 0)),
            pl.BlockSpec((hid, hid), lambda i, s, f: (0, 0)),
            pl.BlockSpec((1, hid), lambda i, s, f: (0, 0)),
            pl.BlockSpec((hid, nclass), lambda i, s, f: (0, 0)),
            pl.BlockSpec((1, nclass), lambda i, s, f: (0, 0)),
        ],
        out_specs=pl.BlockSpec((1, nclass), lambda i, s, f: (0, 0)),
        scratch_shapes=[
            pltpu.VMEM((n_nodes, hid), jnp.float32),
            pltpu.VMEM((node_emb.shape[0], 3 * hid), jnp.float32),
            pltpu.VMEM((_B, 3 * hid), jnp.float32),
            pltpu.VMEM((_B, hid), jnp.float32),
            pltpu.VMEM((_B, hid), jnp.float32),
            pltpu.VMEM((_B, hid), jnp.float32),
        ],
    )
    return pl.pallas_call(
        _scan_body,
        grid_spec=grid_spec,
        out_shape=jax.ShapeDtypeStruct((1, nclass), jnp.float32),
        interpret=interpret,
    )(seqs, flags, node_emb, W_ih.T, b_ih.reshape(1, -1), W_hh.T,
      b_hh.reshape(1, -1), weight, weight_proj.reshape(1, -1),
      out_W.T, out_b.reshape(1, -1))


def kernel(x_index, sequences, embed, weight, weight_proj, W_ih, W_hh, b_ih,
           b_hh, out_W, out_b):
    node_emb = _embedding_mean(x_index, embed)
    seqs = sequences[:, :, 0].T
    flags = _batch_conflict_flags(seqs)
    return _rnn_scan(seqs, flags, node_emb, W_ih, W_hh, b_ih, b_hh, weight,
                     weight_proj, out_W, out_b)
